# baseline (speedup 1.0000x reference)
import functools

import numpy as np
import jax
import jax.numpy as jnp
from jax import lax
from jax.experimental import pallas as pl
from jax.experimental.pallas import tpu as pltpu
from jax.experimental.pallas import tpu_sc as plsc

_ROWS = 16384
_COLS = 200
_DIM = 20
_N = _ROWS * _COLS
_NC = 2
_NS = 16
_NW = _NC * _NS
_PER_W = _N // _NW
_CHUNK = 2048
_NSTEP = _PER_W // _CHUNK
_BLKS = _CHUNK // 16
_LANES = 16


def _body(idx_hbm, table_hbm, out_hbm, table_v, i0, i1, r0, r1, os0, os1):
    wid = lax.axis_index("s") * _NC + lax.axis_index("c")
    base = wid * _PER_W

    pltpu.sync_copy(table_hbm, table_v)

    def compute_chunk(ci, ibuf, rbuf):
        pltpu.sync_copy(idx_hbm.at[pl.ds(base + ci * _CHUNK, _CHUNK)], ibuf)

        def blk(b, carry):
            labels = ibuf[pl.ds(b * _LANES, _LANES)]
            g0 = labels * _DIM
            s0 = b * (_LANES * _DIM) + lax.iota(jnp.int32, _LANES) * _DIM
            for k in range(_DIM):
                vals = plsc.load_gather(table_v, [g0 + k])
                plsc.store_scatter(rbuf, [s0 + k], vals)
            return carry

        lax.fori_loop(0, _BLKS, blk, 0)

    def start_out(ci, rbuf, sem):
        return pltpu.async_copy(
            rbuf, out_hbm.at[pl.ds((base + ci * _CHUNK) * _DIM, _CHUNK * _DIM)],
            sem)

    def drain(rbuf, sem):
        pltpu.make_async_copy(
            rbuf, out_hbm.at[pl.ds(0, _CHUNK * _DIM)], sem).wait()

    compute_chunk(0, i0, r0)
    start_out(0, r0, os0)
    compute_chunk(1, i1, r1)
    start_out(1, r1, os1)

    def pair(g, carry):
        c0 = 2 * g
        drain(r0, os0)
        compute_chunk(c0, i0, r0)
        start_out(c0, r0, os0)
        drain(r1, os1)
        compute_chunk(c0 + 1, i1, r1)
        start_out(c0 + 1, r1, os1)
        return carry

    lax.fori_loop(1, _NSTEP // 2, pair, 0)

    drain(r0, os0)
    drain(r1, os1)


def kernel(speaker_labels, table):
    idx = speaker_labels.reshape(_N)
    grid_kernel = pl.kernel(
        _body,
        out_type=jax.ShapeDtypeStruct((_N * _DIM,), jnp.float32),
        mesh=plsc.VectorSubcoreMesh(
            core_axis_name="c", subcore_axis_name="s",
            num_cores=_NC, num_subcores=_NS,
        ),
        scratch_types=[
            pltpu.VMEM((3 * _DIM,), jnp.float32),
            pltpu.VMEM((_CHUNK,), jnp.int32),
            pltpu.VMEM((_CHUNK,), jnp.int32),
            pltpu.VMEM((_CHUNK * _DIM,), jnp.float32),
            pltpu.VMEM((_CHUNK * _DIM,), jnp.float32),
            pltpu.SemaphoreType.DMA,
            pltpu.SemaphoreType.DMA,
        ],
        compiler_params=pltpu.CompilerParams(use_tc_tiling_on_sc=False, needs_layout_passes=False),
    )
    out = grid_kernel(idx, table.reshape(3 * _DIM))
    return out.reshape(_ROWS, _COLS, _DIM)


# speedup vs baseline: 1.0935x; 1.0935x over previous
import functools

import numpy as np
import jax
import jax.numpy as jnp
from jax import lax
from jax.experimental import pallas as pl
from jax.experimental.pallas import tpu as pltpu
from jax.experimental.pallas import tpu_sc as plsc

_ROWS = 16384
_COLS = 200
_DIM = 20
_N = _ROWS * _COLS
_NC = 2
_NS = 16
_NW = _NC * _NS
_PER_W = _N // _NW
_CHUNK = 2048
_NSTEP = _PER_W // _CHUNK
_BLKS = _CHUNK // 16
_LANES = 16


def _body(idx_hbm, table_hbm, out_hbm, table_v, i0, i1, r0, r1, os0, os1):
    wid = lax.axis_index("s") * _NC + lax.axis_index("c")
    base = wid * _PER_W

    pltpu.sync_copy(table_hbm, table_v)

    def compute_chunk(ci, ibuf, rbuf):
        pltpu.sync_copy(idx_hbm.at[pl.ds(base + ci * _CHUNK, _CHUNK)], ibuf)

        @plsc.parallel_loop(0, _BLKS, unroll=4)
        def blk(b):
            labels = ibuf[pl.ds(b * _LANES, _LANES)]
            g0 = labels * _DIM
            s0 = b * (_LANES * _DIM) + lax.iota(jnp.int32, _LANES) * _DIM
            for k in range(_DIM):
                vals = plsc.load_gather(table_v, [g0 + k])
                plsc.store_scatter(rbuf, [s0 + k], vals)

    def start_out(ci, rbuf, sem):
        return pltpu.async_copy(
            rbuf, out_hbm.at[pl.ds((base + ci * _CHUNK) * _DIM, _CHUNK * _DIM)],
            sem)

    def drain(rbuf, sem):
        pltpu.make_async_copy(
            rbuf, out_hbm.at[pl.ds(0, _CHUNK * _DIM)], sem).wait()

    compute_chunk(0, i0, r0)
    start_out(0, r0, os0)
    compute_chunk(1, i1, r1)
    start_out(1, r1, os1)

    def pair(g, carry):
        c0 = 2 * g
        drain(r0, os0)
        compute_chunk(c0, i0, r0)
        start_out(c0, r0, os0)
        drain(r1, os1)
        compute_chunk(c0 + 1, i1, r1)
        start_out(c0 + 1, r1, os1)
        return carry

    lax.fori_loop(1, _NSTEP // 2, pair, 0)

    drain(r0, os0)
    drain(r1, os1)


def kernel(speaker_labels, table):
    idx = speaker_labels.reshape(_N)
    grid_kernel = pl.kernel(
        _body,
        out_type=jax.ShapeDtypeStruct((_N * _DIM,), jnp.float32),
        mesh=plsc.VectorSubcoreMesh(
            core_axis_name="c", subcore_axis_name="s",
            num_cores=_NC, num_subcores=_NS,
        ),
        scratch_types=[
            pltpu.VMEM((3 * _DIM,), jnp.float32),
            pltpu.VMEM((_CHUNK,), jnp.int32),
            pltpu.VMEM((_CHUNK,), jnp.int32),
            pltpu.VMEM((_CHUNK * _DIM,), jnp.float32),
            pltpu.VMEM((_CHUNK * _DIM,), jnp.float32),
            pltpu.SemaphoreType.DMA,
            pltpu.SemaphoreType.DMA,
        ],
        compiler_params=pltpu.CompilerParams(use_tc_tiling_on_sc=False, needs_layout_passes=False),
    )
    out = grid_kernel(idx, table.reshape(3 * _DIM))
    return out.reshape(_ROWS, _COLS, _DIM)


# speedup vs baseline: 29.1735x; 26.6791x over previous
import functools

import numpy as np
import jax
import jax.numpy as jnp
from jax import lax
from jax.experimental import pallas as pl
from jax.experimental.pallas import tpu as pltpu
from jax.experimental.pallas import tpu_sc as plsc

_ROWS = 16384
_COLS = 200
_DIM = 20
_NC = 2
_NS = 16
_NW = _NC * _NS
_TJ = _COLS // 8
_TI = _ROWS // 128
_TI_W = _TI // _NW
_PAIR_I = 256
_LANES = 16
_NPAIRS = _TJ


def _body(lbl_hbm, table_hbm, out_hbm, table_v, l0, l1, r0, r1, s0, s1):
    wid = lax.axis_index("s") * _NC + lax.axis_index("c")
    ti0 = wid * _TI_W
    i_base = ti0 * 128

    pltpu.sync_copy(table_hbm, table_v)

    def compute_step(tj, q, lbuf, rbuf):
        pltpu.sync_copy(
            lbl_hbm.at[pl.ds(tj * 8, 8), pl.ds(i_base + q * _PAIR_I, _PAIR_I)],
            lbuf)

        @plsc.parallel_loop(0, 8 * (_PAIR_I // _LANES), unroll=2)
        def chunk(p):
            sj = lax.shift_right_logical(p, 4)
            c = lax.bitwise_and(p, 15)
            tix = lax.shift_right_logical(c, 3)
            cm8 = lax.bitwise_and(c, 7)
            lbl = lbuf[sj, pl.ds(c * _LANES, _LANES)]
            g0 = lbl * _DIM
            for k in range(_DIM):
                vals = plsc.load_gather(table_v, [g0 + k])
                rbuf[k, tix, sj, pl.ds(cm8 * _LANES, _LANES)] = vals

    def start_out(tj, q, rbuf, sem):
        return pltpu.async_copy(
            rbuf,
            out_hbm.at[:, tj, pl.ds(ti0 + 2 * q, 2), :, :],
            sem)

    def drain(rbuf, sem):
        pltpu.make_async_copy(
            rbuf, out_hbm.at[:, 0, pl.ds(0, 2), :, :], sem).wait()

    compute_step(0, 0, l0, r0)
    start_out(0, 0, r0, s0)
    compute_step(0, 1, l1, r1)
    start_out(0, 1, r1, s1)

    def pair(tj, carry):
        drain(r0, s0)
        compute_step(tj, 0, l0, r0)
        start_out(tj, 0, r0, s0)
        drain(r1, s1)
        compute_step(tj, 1, l1, r1)
        start_out(tj, 1, r1, s1)
        return carry

    lax.fori_loop(1, _NPAIRS, pair, 0)

    drain(r0, s0)
    drain(r1, s1)


def kernel(speaker_labels, table):
    lbl_t = speaker_labels.T
    grid_kernel = pl.kernel(
        _body,
        out_type=jax.ShapeDtypeStruct((_DIM, _TJ, _TI, 8, 128), jnp.float32),
        mesh=plsc.VectorSubcoreMesh(
            core_axis_name="c", subcore_axis_name="s",
            num_cores=_NC, num_subcores=_NS,
        ),
        scratch_types=[
            pltpu.VMEM((3 * _DIM,), jnp.float32),
            pltpu.VMEM((8, _PAIR_I), jnp.int32),
            pltpu.VMEM((8, _PAIR_I), jnp.int32),
            pltpu.VMEM((_DIM, 2, 8, 128), jnp.float32),
            pltpu.VMEM((_DIM, 2, 8, 128), jnp.float32),
            pltpu.SemaphoreType.DMA,
            pltpu.SemaphoreType.DMA,
        ],
        compiler_params=pltpu.CompilerParams(
            use_tc_tiling_on_sc=False, needs_layout_passes=False),
    )
    out_t = grid_kernel(lbl_t, table.reshape(3 * _DIM))
    return out_t.transpose(2, 4, 1, 3, 0).reshape(_ROWS, _COLS, _DIM)
